# baseline (device time: 106546 ns/iter reference)
import jax
import jax.numpy as jnp
from jax import lax
from jax.experimental import pallas as pl
from jax.experimental.pallas import tpu as pltpu

T = 2048
D = 4096
V_LOC = 8192
Q = V_LOC // 4
V_TILE = 512
N_TILES = Q // V_TILE


def _cast_body(x_ref, xb_ref):
    xb_ref[...] = x_ref[...].astype(jnp.bfloat16)


def _stats_body(q_ref, x_ref, w_ref, lab_ref, acc_ref):
    j = pl.program_id(0)
    my_x = lax.axis_index("x")

    @pl.when(j == 0)
    def _():
        acc_ref[...] = jnp.zeros((T, 8), jnp.float32)
        acc_ref[:, 0:1] = jnp.full((T, 1), -jnp.inf, jnp.float32)

    logits = jnp.dot(
        x_ref[...],
        w_ref[...].astype(jnp.bfloat16),
        preferred_element_type=jnp.float32,
    )

    tile_m = jnp.max(logits, axis=1, keepdims=True)
    m_prev = acc_ref[:, 0:1]
    m_new = jnp.maximum(m_prev, tile_m)
    acc_ref[:, 1:2] = acc_ref[:, 1:2] * jnp.exp(m_prev - m_new) + jnp.sum(
        jnp.exp(logits - m_new), axis=1, keepdims=True
    )
    acc_ref[:, 0:1] = m_new

    col0 = my_x * V_LOC + q_ref[0] * Q + j * V_TILE
    cols = col0 + lax.broadcasted_iota(jnp.int32, (T, V_TILE), 1)
    hit = cols == lab_ref[...]
    acc_ref[:, 2:3] += jnp.sum(jnp.where(hit, logits, 0.0), axis=1, keepdims=True)


def _allreduce_body(stats_ref, out_ref, send_ref, recv_ref, send_sems, recv_sems):
    my_x = lax.axis_index("x")
    my_y = lax.axis_index("y")
    my_z = lax.axis_index("z")
    partners = [
        (1 - my_x, my_y, my_z),
        (my_x, 1 - my_y, my_z),
        (my_x, my_y, 1 - my_z),
    ]

    rowstats = jnp.transpose(stats_ref[...], (1, 0))

    barrier_sem = pltpu.get_barrier_semaphore()
    for p in partners:
        pl.semaphore_signal(
            barrier_sem, inc=1, device_id=p,
            device_id_type=pl.DeviceIdType.MESH,
        )
    pl.semaphore_wait(barrier_sem, 3)

    m = rowstats[0:1, :]
    l = rowstats[1:2, :]
    g = rowstats[2:3, :]

    for k, p in enumerate(partners):
        send_ref[0:1, :] = m
        send_ref[1:2, :] = l
        send_ref[2:3, :] = g
        rdma = pltpu.make_async_remote_copy(
            src_ref=send_ref,
            dst_ref=recv_ref.at[k],
            send_sem=send_sems.at[k],
            recv_sem=recv_sems.at[k],
            device_id=p,
            device_id_type=pl.DeviceIdType.MESH,
        )
        rdma.start()
        rdma.wait()

        pm = recv_ref[k, 0:1, :]
        pl_ = recv_ref[k, 1:2, :]
        pg = recv_ref[k, 2:3, :]
        m_new = jnp.maximum(m, pm)
        l = l * jnp.exp(m - m_new) + pl_ * jnp.exp(pm - m_new)
        g = g + pg
        m = m_new

    out_ref[...] = (m + jnp.log(l)) - g


def kernel(x, W, labels):
    q = (lax.axis_index("y") * 2 + lax.axis_index("z")).astype(jnp.int32)

    xb = pl.pallas_call(
        _cast_body,
        grid=(8,),
        in_specs=[pl.BlockSpec((T // 8, D), lambda i: (i, 0))],
        out_specs=pl.BlockSpec((T // 8, D), lambda i: (i, 0)),
        out_shape=jax.ShapeDtypeStruct((T, D), jnp.bfloat16),
    )(x)

    stats = pl.pallas_call(
        _stats_body,
        grid_spec=pltpu.PrefetchScalarGridSpec(
            num_scalar_prefetch=1,
            grid=(N_TILES,),
            in_specs=[
                pl.BlockSpec((T, D), lambda j, q_ref: (0, 0)),
                pl.BlockSpec(
                    (D, V_TILE), lambda j, q_ref: (0, q_ref[0] * N_TILES + j)
                ),
                pl.BlockSpec((T, 1), lambda j, q_ref: (0, 0)),
            ],
            out_specs=pl.BlockSpec((T, 8), lambda j, q_ref: (0, 0)),
        ),
        out_shape=jax.ShapeDtypeStruct((T, 8), jnp.float32),
        compiler_params=pltpu.CompilerParams(
            vmem_limit_bytes=128 * 1024 * 1024,
        ),
    )(q.reshape(1), xb, W, labels.reshape(T, 1))

    nll = pl.pallas_call(
        _allreduce_body,
        in_specs=[pl.BlockSpec(memory_space=pltpu.VMEM)],
        out_specs=pl.BlockSpec(memory_space=pltpu.VMEM),
        out_shape=jax.ShapeDtypeStruct((1, T), jnp.float32),
        scratch_shapes=[
            pltpu.VMEM((8, T), jnp.float32),
            pltpu.VMEM((3, 8, T), jnp.float32),
            pltpu.SemaphoreType.DMA((3,)),
            pltpu.SemaphoreType.DMA((3,)),
        ],
        compiler_params=pltpu.CompilerParams(collective_id=0),
    )(stats)

    return nll.reshape(T)


# device time: 93543 ns/iter; 1.1390x vs baseline; 1.1390x over previous
import jax
import jax.numpy as jnp
from jax import lax
from jax.experimental import pallas as pl
from jax.experimental.pallas import tpu as pltpu

T = 2048
D = 4096
V_LOC = 8192
Q = V_LOC // 4
K_TILE = 512
N_K = D // K_TILE
EPI_CHUNK = 512


def _cast_body(x_ref, xb_ref):
    xb_ref[...] = x_ref[...].astype(jnp.bfloat16)


def _stats_body(q_ref, x_ref, w_ref, lab_ref, acc_ref, logits_ref):
    k = pl.program_id(0)
    my_x = lax.axis_index("x")

    partial = jnp.dot(
        x_ref[...],
        w_ref[...].astype(jnp.bfloat16),
        preferred_element_type=jnp.float32,
    )

    @pl.when(k == 0)
    def _():
        logits_ref[...] = partial

    @pl.when(k > 0)
    def _():
        logits_ref[...] += partial

    @pl.when(k == N_K - 1)
    def _():
        col0 = my_x * V_LOC + q_ref[0] * Q
        m = jnp.full((T, 1), -jnp.inf, jnp.float32)
        l = jnp.zeros((T, 1), jnp.float32)
        g = jnp.zeros((T, 1), jnp.float32)
        for c in range(Q // EPI_CHUNK):
            lg = logits_ref[:, c * EPI_CHUNK:(c + 1) * EPI_CHUNK]
            tile_m = jnp.max(lg, axis=1, keepdims=True)
            m_new = jnp.maximum(m, tile_m)
            l = l * jnp.exp(m - m_new) + jnp.sum(
                jnp.exp(lg - m_new), axis=1, keepdims=True
            )
            cols = (col0 + c * EPI_CHUNK) + lax.broadcasted_iota(
                jnp.int32, (T, EPI_CHUNK), 1
            )
            g += jnp.sum(
                jnp.where(cols == lab_ref[...], lg, 0.0), axis=1, keepdims=True
            )
            m = m_new
        acc_ref[...] = jnp.zeros((T, 8), jnp.float32)
        acc_ref[:, 0:1] = m
        acc_ref[:, 1:2] = l
        acc_ref[:, 2:3] = g


def _allreduce_body(stats_ref, out_ref, send_ref, recv_ref, send_sems, recv_sems):
    my_x = lax.axis_index("x")
    my_y = lax.axis_index("y")
    my_z = lax.axis_index("z")
    partners = [
        (1 - my_x, my_y, my_z),
        (my_x, 1 - my_y, my_z),
        (my_x, my_y, 1 - my_z),
    ]

    barrier_sem = pltpu.get_barrier_semaphore()
    for p in partners:
        pl.semaphore_signal(
            barrier_sem, inc=1, device_id=p,
            device_id_type=pl.DeviceIdType.MESH,
        )
    pl.semaphore_wait(barrier_sem, 3)

    rowstats = jnp.transpose(stats_ref[...], (1, 0))
    m = rowstats[0:1, :]
    l = rowstats[1:2, :]
    g = rowstats[2:3, :]

    for k, p in enumerate(partners):
        send_ref[0:1, :] = m
        send_ref[1:2, :] = l
        send_ref[2:3, :] = g
        rdma = pltpu.make_async_remote_copy(
            src_ref=send_ref,
            dst_ref=recv_ref.at[k],
            send_sem=send_sems.at[k],
            recv_sem=recv_sems.at[k],
            device_id=p,
            device_id_type=pl.DeviceIdType.MESH,
        )
        rdma.start()
        rdma.wait()

        pm = recv_ref[k, 0:1, :]
        pl_ = recv_ref[k, 1:2, :]
        pg = recv_ref[k, 2:3, :]
        m_new = jnp.maximum(m, pm)
        l = l * jnp.exp(m - m_new) + pl_ * jnp.exp(pm - m_new)
        g = g + pg
        m = m_new

    out_ref[...] = (m + jnp.log(l)) - g


def kernel(x, W, labels):
    q = (lax.axis_index("y") * 2 + lax.axis_index("z")).astype(jnp.int32)

    xb = pl.pallas_call(
        _cast_body,
        grid=(8,),
        in_specs=[pl.BlockSpec((T // 8, D), lambda i: (i, 0))],
        out_specs=pl.BlockSpec((T // 8, D), lambda i: (i, 0)),
        out_shape=jax.ShapeDtypeStruct((T, D), jnp.bfloat16),
    )(x)

    stats = pl.pallas_call(
        _stats_body,
        grid_spec=pltpu.PrefetchScalarGridSpec(
            num_scalar_prefetch=1,
            grid=(N_K,),
            in_specs=[
                pl.BlockSpec((T, K_TILE), lambda k, q_ref: (0, k)),
                pl.BlockSpec((K_TILE, Q), lambda k, q_ref: (k, q_ref[0])),
                pl.BlockSpec((T, 1), lambda k, q_ref: (0, 0)),
            ],
            out_specs=pl.BlockSpec((T, 8), lambda k, q_ref: (0, 0)),
            scratch_shapes=[
                pltpu.VMEM((T, Q), jnp.float32),
            ],
        ),
        out_shape=jax.ShapeDtypeStruct((T, 8), jnp.float32),
        compiler_params=pltpu.CompilerParams(
            vmem_limit_bytes=128 * 1024 * 1024,
        ),
    )(q.reshape(1), xb, W, labels.reshape(T, 1))

    nll = pl.pallas_call(
        _allreduce_body,
        in_specs=[pl.BlockSpec(memory_space=pltpu.VMEM)],
        out_specs=pl.BlockSpec(memory_space=pltpu.VMEM),
        out_shape=jax.ShapeDtypeStruct((1, T), jnp.float32),
        scratch_shapes=[
            pltpu.VMEM((8, T), jnp.float32),
            pltpu.VMEM((3, 8, T), jnp.float32),
            pltpu.SemaphoreType.DMA((3,)),
            pltpu.SemaphoreType.DMA((3,)),
        ],
        compiler_params=pltpu.CompilerParams(collective_id=0),
    )(stats)

    return nll.reshape(T)


# device time: 75064 ns/iter; 1.4194x vs baseline; 1.2462x over previous
import jax
import jax.numpy as jnp
from jax import lax
from jax.experimental import pallas as pl
from jax.experimental.pallas import tpu as pltpu

T = 2048
D = 4096
V_LOC = 8192
Q = V_LOC // 4
K_TILE = 512
N_K = D // K_TILE
EPI_CHUNK = 512


def _stats_body(q_ref, x_ref, w_ref, lab_ref, acc_ref, logits_ref):
    k = pl.program_id(0)
    my_x = lax.axis_index("x")

    partial = jnp.dot(
        x_ref[...].astype(jnp.bfloat16),
        w_ref[...].astype(jnp.bfloat16),
        preferred_element_type=jnp.float32,
    )

    @pl.when(k == 0)
    def _():
        logits_ref[...] = partial

    @pl.when(k > 0)
    def _():
        logits_ref[...] += partial

    @pl.when(k == N_K - 1)
    def _():
        col0 = my_x * V_LOC + q_ref[0] * Q
        lab_col = jnp.transpose(lab_ref[...], (1, 0))
        m = jnp.full((T, 1), -jnp.inf, jnp.float32)
        l = jnp.zeros((T, 1), jnp.float32)
        g = jnp.zeros((T, 1), jnp.float32)
        for c in range(Q // EPI_CHUNK):
            lg = logits_ref[:, c * EPI_CHUNK:(c + 1) * EPI_CHUNK]
            tile_m = jnp.max(lg, axis=1, keepdims=True)
            m_new = jnp.maximum(m, tile_m)
            l = l * jnp.exp(m - m_new) + jnp.sum(
                jnp.exp(lg - m_new), axis=1, keepdims=True
            )
            cols = (col0 + c * EPI_CHUNK) + lax.broadcasted_iota(
                jnp.int32, (T, EPI_CHUNK), 1
            )
            g += jnp.sum(
                jnp.where(cols == lab_col, lg, 0.0), axis=1, keepdims=True
            )
            m = m_new
        acc_ref[...] = jnp.zeros((T, 8), jnp.float32)
        acc_ref[:, 0:1] = m
        acc_ref[:, 1:2] = l
        acc_ref[:, 2:3] = g


def _allreduce_body(stats_ref, out_ref, send_ref, recv_ref, send_sems, recv_sems):
    my_x = lax.axis_index("x")
    my_y = lax.axis_index("y")
    my_z = lax.axis_index("z")
    partners = [
        (1 - my_x, my_y, my_z),
        (my_x, 1 - my_y, my_z),
        (my_x, my_y, 1 - my_z),
    ]

    barrier_sem = pltpu.get_barrier_semaphore()
    for p in partners:
        pl.semaphore_signal(
            barrier_sem, inc=1, device_id=p,
            device_id_type=pl.DeviceIdType.MESH,
        )
    pl.semaphore_wait(barrier_sem, 3)

    rowstats = jnp.transpose(stats_ref[...], (1, 0))
    m = rowstats[0:1, :]
    l = rowstats[1:2, :]
    g = rowstats[2:3, :]

    for k, p in enumerate(partners):
        send_ref[0:1, :] = m
        send_ref[1:2, :] = l
        send_ref[2:3, :] = g
        rdma = pltpu.make_async_remote_copy(
            src_ref=send_ref,
            dst_ref=recv_ref.at[k],
            send_sem=send_sems.at[k],
            recv_sem=recv_sems.at[k],
            device_id=p,
            device_id_type=pl.DeviceIdType.MESH,
        )
        rdma.start()
        rdma.wait()

        pm = recv_ref[k, 0:1, :]
        pl_ = recv_ref[k, 1:2, :]
        pg = recv_ref[k, 2:3, :]
        m_new = jnp.maximum(m, pm)
        l = l * jnp.exp(m - m_new) + pl_ * jnp.exp(pm - m_new)
        g = g + pg
        m = m_new

    out_ref[...] = (m + jnp.log(l)) - g


def kernel(x, W, labels):
    q = (lax.axis_index("y") * 2 + lax.axis_index("z")).astype(jnp.int32)

    stats = pl.pallas_call(
        _stats_body,
        grid_spec=pltpu.PrefetchScalarGridSpec(
            num_scalar_prefetch=1,
            grid=(N_K,),
            in_specs=[
                pl.BlockSpec((T, K_TILE), lambda k, q_ref: (0, k)),
                pl.BlockSpec((K_TILE, Q), lambda k, q_ref: (k, q_ref[0])),
                pl.BlockSpec((1, T), lambda k, q_ref: (0, 0)),
            ],
            out_specs=pl.BlockSpec((T, 8), lambda k, q_ref: (0, 0)),
            scratch_shapes=[
                pltpu.VMEM((T, Q), jnp.float32),
            ],
        ),
        out_shape=jax.ShapeDtypeStruct((T, 8), jnp.float32),
        compiler_params=pltpu.CompilerParams(
            vmem_limit_bytes=128 * 1024 * 1024,
        ),
    )(q.reshape(1), x, W, labels.reshape(1, T))

    nll = pl.pallas_call(
        _allreduce_body,
        in_specs=[pl.BlockSpec(memory_space=pltpu.VMEM)],
        out_specs=pl.BlockSpec(memory_space=pltpu.VMEM),
        out_shape=jax.ShapeDtypeStruct((1, T), jnp.float32),
        scratch_shapes=[
            pltpu.VMEM((8, T), jnp.float32),
            pltpu.VMEM((3, 8, T), jnp.float32),
            pltpu.SemaphoreType.DMA((3,)),
            pltpu.SemaphoreType.DMA((3,)),
        ],
        compiler_params=pltpu.CompilerParams(collective_id=0),
    )(stats)

    return nll.reshape(T)


# device time: 72762 ns/iter; 1.4643x vs baseline; 1.0316x over previous
import jax
import jax.numpy as jnp
from jax import lax
from jax.experimental import pallas as pl
from jax.experimental.pallas import tpu as pltpu

T = 2048
D = 4096
V_LOC = 8192
Q = V_LOC // 4
K_TILE = 512
N_K = D // K_TILE
EPI_CHUNK = 512


_PEER_MASKS = [
    (1, 0, 0), (0, 1, 0), (0, 0, 1),
    (1, 1, 0), (1, 0, 1), (0, 1, 1), (1, 1, 1),
]


def _stats_body(
    q_ref, x_ref, w_ref, lab_ref, out_ref,
    logits_ref, send_ref, recv_ref, send_sems, recv_sems,
):
    k = pl.program_id(0)
    my_x = lax.axis_index("x")
    my_y = lax.axis_index("y")
    my_z = lax.axis_index("z")

    partial = jnp.dot(
        x_ref[...].astype(jnp.bfloat16),
        w_ref[...].astype(jnp.bfloat16),
        preferred_element_type=jnp.float32,
    )

    @pl.when(k == 0)
    def _():
        logits_ref[...] = partial

    @pl.when(k > 0)
    def _():
        logits_ref[...] += partial

    @pl.when(k == N_K - 1)
    def _():
        col0 = my_x * V_LOC + q_ref[0] * Q
        lab_col = jnp.transpose(lab_ref[...], (1, 0))
        m = jnp.full((T, 1), -jnp.inf, jnp.float32)
        l = jnp.zeros((T, 1), jnp.float32)
        g = jnp.zeros((T, 1), jnp.float32)
        for c in range(Q // EPI_CHUNK):
            lg = logits_ref[:, c * EPI_CHUNK:(c + 1) * EPI_CHUNK]
            tile_m = jnp.max(lg, axis=1, keepdims=True)
            m_new = jnp.maximum(m, tile_m)
            l = l * jnp.exp(m - m_new) + jnp.sum(
                jnp.exp(lg - m_new), axis=1, keepdims=True
            )
            cols = (col0 + c * EPI_CHUNK) + lax.broadcasted_iota(
                jnp.int32, (T, EPI_CHUNK), 1
            )
            g += jnp.sum(
                jnp.where(cols == lab_col, lg, 0.0), axis=1, keepdims=True
            )
            m = m_new

        m = jnp.transpose(m, (1, 0))
        l = jnp.transpose(l, (1, 0))
        g = jnp.transpose(g, (1, 0))
        send_ref[0:1, :] = m
        send_ref[1:2, :] = l
        send_ref[2:3, :] = g

        peers = [
            (
                my_x + dx - 2 * dx * my_x,
                my_y + dy - 2 * dy * my_y,
                my_z + dz - 2 * dz * my_z,
            )
            for dx, dy, dz in _PEER_MASKS
        ]

        barrier_sem = pltpu.get_barrier_semaphore()
        for p in peers:
            pl.semaphore_signal(
                barrier_sem, inc=1, device_id=p,
                device_id_type=pl.DeviceIdType.MESH,
            )
        pl.semaphore_wait(barrier_sem, 7)

        rdmas = []
        for i, p in enumerate(peers):
            rdma = pltpu.make_async_remote_copy(
                src_ref=send_ref,
                dst_ref=recv_ref.at[i],
                send_sem=send_sems.at[i],
                recv_sem=recv_sems.at[i],
                device_id=p,
                device_id_type=pl.DeviceIdType.MESH,
            )
            rdma.start()
            rdmas.append(rdma)

        for i, rdma in enumerate(rdmas):
            rdma.wait()
            pm = recv_ref[i, 0:1, :]
            pl_ = recv_ref[i, 1:2, :]
            pg = recv_ref[i, 2:3, :]
            m_new = jnp.maximum(m, pm)
            l = l * jnp.exp(m - m_new) + pl_ * jnp.exp(pm - m_new)
            g = g + pg
            m = m_new

        out_ref[...] = (m + jnp.log(l)) - g


def kernel(x, W, labels):
    q = (lax.axis_index("y") * 2 + lax.axis_index("z")).astype(jnp.int32)

    nll = pl.pallas_call(
        _stats_body,
        grid_spec=pltpu.PrefetchScalarGridSpec(
            num_scalar_prefetch=1,
            grid=(N_K,),
            in_specs=[
                pl.BlockSpec((T, K_TILE), lambda k, q_ref: (0, k)),
                pl.BlockSpec((K_TILE, Q), lambda k, q_ref: (k, q_ref[0])),
                pl.BlockSpec((1, T), lambda k, q_ref: (0, 0)),
            ],
            out_specs=pl.BlockSpec((1, T), lambda k, q_ref: (0, 0)),
            scratch_shapes=[
                pltpu.VMEM((T, Q), jnp.float32),
                pltpu.VMEM((8, T), jnp.float32),
                pltpu.VMEM((7, 8, T), jnp.float32),
                pltpu.SemaphoreType.DMA((7,)),
                pltpu.SemaphoreType.DMA((7,)),
            ],
        ),
        out_shape=jax.ShapeDtypeStruct((1, T), jnp.float32),
        compiler_params=pltpu.CompilerParams(
            vmem_limit_bytes=128 * 1024 * 1024,
            collective_id=0,
        ),
    )(q.reshape(1), x, W, labels.reshape(1, T))

    return nll.reshape(T)
